# Initial kernel scaffold; baseline (speedup 1.0000x reference)
import jax
import jax.numpy as jnp
from jax.experimental import pallas as pl


def kernel(poi_vec, category_embeddings, single_embeddings):
    raise NotImplementedError("write your pallas kernel here")


# speedup vs baseline: 2.6298x; 2.6298x over previous
import functools

import jax
import jax.numpy as jnp
from jax import lax
from jax.experimental import pallas as pl
from jax.experimental.pallas import tpu as pltpu
from jax.experimental.pallas import tpu_sc as plsc

_B = 16384
_D = 64
_LANES = 16

_INFO = plsc.get_sparse_core_info()
_NC = _INFO.num_cores
_NS = _INFO.num_subcores
_NW = _NC * _NS
_BPW = _B // _NW
_CHUNK = 128
_NCHUNK = _BPW // _CHUNK


def _body(pidx_hbm, cidx_hbm, single_hbm, cat_hbm, out_hbm,
          idxp_v, idxc_v, rows_a, rows_b, sem_a, sem_b):
    wid = lax.axis_index("s") * _NC + lax.axis_index("c")
    base = wid * _BPW

    pltpu.sync_copy(pidx_hbm.at[pl.ds(base, _BPW)], idxp_v)
    pltpu.sync_copy(cidx_hbm.at[pl.ds(base, _BPW)], idxc_v)

    copies = []
    for j in range(_NCHUNK):
        sl = pl.ds(j * _CHUNK, _CHUNK)
        copies.append(pltpu.async_copy(
            single_hbm.at[idxp_v.at[sl]], rows_a.at[sl], sem_a))
        copies.append(pltpu.async_copy(
            cat_hbm.at[idxc_v.at[sl]], rows_b.at[sl], sem_b))
    for cp in copies:
        cp.wait()

    def add_row(i, carry):
        for j in range(_D // _LANES):
            sl = pl.ds(j * _LANES, _LANES)
            rows_a[i, sl] = rows_a[i, sl] + rows_b[i, sl]
        return carry
    lax.fori_loop(0, _BPW, add_row, 0)

    pltpu.sync_copy(rows_a, out_hbm.at[pl.ds(base, _BPW)])


@jax.jit
def _poi_embedding(pidx, cidx, single_embeddings, category_embeddings):
    mesh = plsc.VectorSubcoreMesh(core_axis_name="c", subcore_axis_name="s")
    kfn = pl.kernel(
        _body,
        out_type=jax.ShapeDtypeStruct((_B, _D), jnp.float32),
        mesh=mesh,
        scratch_types=[
            pltpu.VMEM((_BPW,), jnp.int32),
            pltpu.VMEM((_BPW,), jnp.int32),
            pltpu.VMEM((_BPW, _D), jnp.float32),
            pltpu.VMEM((_BPW, _D), jnp.float32),
            pltpu.SemaphoreType.DMA,
            pltpu.SemaphoreType.DMA,
        ],
        compiler_params=pltpu.CompilerParams(use_tc_tiling_on_sc=False),
    )
    return kfn(pidx, cidx, single_embeddings, category_embeddings)


def kernel(poi_vec, category_embeddings, single_embeddings):
    pv = poi_vec.T
    poi_index = pv[0]
    cate_index = pv[1]
    return _poi_embedding(poi_index, cate_index,
                          single_embeddings, category_embeddings)


# speedup vs baseline: 2.6941x; 1.0244x over previous
import functools

import jax
import jax.numpy as jnp
from jax import lax
from jax.experimental import pallas as pl
from jax.experimental.pallas import tpu as pltpu
from jax.experimental.pallas import tpu_sc as plsc

_B = 16384
_D = 64
_LANES = 16

_INFO = plsc.get_sparse_core_info()
_NC = _INFO.num_cores
_NS = _INFO.num_subcores
_NW = _NC * _NS
_BPW = _B // _NW
_CHUNK = 128
_NCHUNK = _BPW // _CHUNK


def _body(pidx_hbm, cidx_hbm, single_hbm, cat_hbm, out_hbm,
          idxp_v, idxc_v, rows_a, rows_b,
          sem_idx, sem_w, *sems_ab):
    sems_a = sems_ab[:_NCHUNK]
    sems_b = sems_ab[_NCHUNK:]
    wid = lax.axis_index("s") * _NC + lax.axis_index("c")
    base = wid * _BPW

    cp0 = pltpu.async_copy(pidx_hbm.at[pl.ds(base, _BPW)], idxp_v, sem_idx)
    cp1 = pltpu.async_copy(cidx_hbm.at[pl.ds(base, _BPW)], idxc_v, sem_idx)
    cp0.wait()
    cp1.wait()

    gathers = []
    for j in range(_NCHUNK):
        sl = pl.ds(j * _CHUNK, _CHUNK)
        gathers.append((
            pltpu.async_copy(single_hbm.at[idxp_v.at[sl]], rows_a.at[sl],
                             sems_a[j]),
            pltpu.async_copy(cat_hbm.at[idxc_v.at[sl]], rows_b.at[sl],
                             sems_b[j]),
        ))

    writes = []
    for j in range(_NCHUNK):
        ga, gb = gathers[j]
        ga.wait()
        gb.wait()

        def add_row(i, carry):
            for t in range(_D // _LANES):
                sl = pl.ds(t * _LANES, _LANES)
                rows_a[i, sl] = rows_a[i, sl] + rows_b[i, sl]
            return carry
        lax.fori_loop(j * _CHUNK, (j + 1) * _CHUNK, add_row, 0)

        writes.append(pltpu.async_copy(
            rows_a.at[pl.ds(j * _CHUNK, _CHUNK)],
            out_hbm.at[pl.ds(base + j * _CHUNK, _CHUNK)],
            sem_w))
    for cp in writes:
        cp.wait()


@jax.jit
def _poi_embedding(pidx, cidx, single_embeddings, category_embeddings):
    mesh = plsc.VectorSubcoreMesh(core_axis_name="c", subcore_axis_name="s")
    kfn = pl.kernel(
        _body,
        out_type=jax.ShapeDtypeStruct((_B, _D), jnp.float32),
        mesh=mesh,
        scratch_types=[
            pltpu.VMEM((_BPW,), jnp.int32),
            pltpu.VMEM((_BPW,), jnp.int32),
            pltpu.VMEM((_BPW, _D), jnp.float32),
            pltpu.VMEM((_BPW, _D), jnp.float32),
            pltpu.SemaphoreType.DMA,
            pltpu.SemaphoreType.DMA,
        ] + [pltpu.SemaphoreType.DMA] * (2 * _NCHUNK),
        compiler_params=pltpu.CompilerParams(use_tc_tiling_on_sc=False),
    )
    return kfn(pidx, cidx, single_embeddings, category_embeddings)


def kernel(poi_vec, category_embeddings, single_embeddings):
    pv = poi_vec.T
    poi_index = pv[0]
    cate_index = pv[1]
    return _poi_embedding(poi_index, cate_index,
                          single_embeddings, category_embeddings)
